# baseline (speedup 1.0000x reference)
import functools

import jax
import jax.numpy as jnp
from jax import lax
from jax.experimental import pallas as pl
from jax.experimental.pallas import tpu as pltpu
from jax.experimental.pallas import tpu_sc as plsc

N = 10000
D = 128
NC = 2
NS = 16
CHUNK = 128
ACC_ROWS = 10112
BLK = 1000
GRID = N // BLK


def _leaky(v):
    return jnp.where(v >= 0, v, 0.01 * v)


def _mm_t(a, b):
    return lax.dot_general(a, b, (((1,), (1,)), ((), ())),
                           preferred_element_type=jnp.float32)


def _make_seg_sum(cpt0, cpt1):
    mesh = plsc.VectorSubcoreMesh(core_axis_name="c", subcore_axis_name="s")

    @functools.partial(
        pl.kernel,
        out_type=jax.ShapeDtypeStruct((NC, ACC_ROWS, D), jnp.float32),
        mesh=mesh,
        scratch_types=[
            pltpu.VMEM((CHUNK,), jnp.int32),
            pltpu.VMEM((CHUNK,), jnp.int32),
            pltpu.VMEM((CHUNK, D), jnp.float32),
            pltpu.VMEM_SHARED((ACC_ROWS, D), jnp.float32),
            pltpu.SemaphoreType.DMA,
        ],
    )
    def seg_sum(x_hbm, src_hbm, dst_hbm, zeros_hbm, out_hbm,
                sbuf, dbuf, rows, acc, gsem):
        c = lax.axis_index("c")
        s = lax.axis_index("s")
        cpt = jnp.where(c == 0, cpt0, cpt1)
        tbase = jnp.where(c == 0, s * cpt0, NS * cpt0 + s * cpt1)
        ebase = tbase * CHUNK

        zr = ACC_ROWS // NS
        pltpu.sync_copy(zeros_hbm.at[pl.ds(s * zr, zr)],
                        acc.at[pl.ds(s * zr, zr)])
        plsc.subcore_barrier()

        def body(i, carry):
            off = ebase + i * CHUNK
            pltpu.sync_copy(src_hbm.at[pl.ds(off, CHUNK)], sbuf)
            pltpu.sync_copy(dst_hbm.at[pl.ds(off, CHUNK)], dbuf)
            pltpu.async_copy(x_hbm.at[sbuf], rows, gsem).wait()
            pltpu.sync_copy(rows, acc.at[dbuf], add=True)
            return carry

        lax.fori_loop(0, cpt, body, 0)

        plsc.subcore_barrier()

        orow = ACC_ROWS // NS
        pltpu.sync_copy(acc.at[pl.ds(s * orow, orow)],
                        out_hbm.at[c, pl.ds(s * orow, orow)])

    return seg_sum


_row_spec = pl.BlockSpec((BLK, D), lambda i: (i, 0))
_w_spec = pl.BlockSpec((D, D), lambda i: (0, 0))
_b_spec = pl.BlockSpec((1, D), lambda i: (0, 0))


def _tc_first_body(x_ref, id_ref, lw_ref, lb_ref, x1_ref, u1_ref):
    xb = x_ref[...]
    ss = jnp.sum(xb * xb, axis=1, keepdims=True)
    xn = xb / jnp.maximum(jnp.sqrt(ss), 1e-12)
    x1_ref[...] = xn
    u1_ref[...] = _leaky(_mm_t(xn, lw_ref[...]) + lb_ref[...]) + id_ref[...]


_tc_first = pl.pallas_call(
    _tc_first_body,
    grid=(GRID,),
    in_specs=[_row_spec, _row_spec, _w_spec, _b_spec],
    out_specs=[_row_spec, _row_spec],
    out_shape=[jax.ShapeDtypeStruct((N, D), jnp.float32),
               jax.ShapeDtypeStruct((N, D), jnp.float32)],
)


def _tc_mid_body(s0_ref, s1_ref, u_ref, id_ref, w_ref, gw_ref, gb_ref,
                 lw_ref, lb_ref, x_ref, un_ref):
    sb = s0_ref[...] + s1_ref[...]
    h = _leaky(_mm_t(sb, w_ref[...]))
    x = _leaky(_mm_t(h, gw_ref[...]) + gb_ref[...] + u_ref[...])
    x_ref[...] = x
    un_ref[...] = _leaky(_mm_t(x, lw_ref[...]) + lb_ref[...]) + id_ref[...]


_tc_mid = pl.pallas_call(
    _tc_mid_body,
    grid=(GRID,),
    in_specs=[_row_spec, _row_spec, _row_spec, _row_spec,
              _w_spec, _w_spec, _b_spec, _w_spec, _b_spec],
    out_specs=[_row_spec, _row_spec],
    out_shape=[jax.ShapeDtypeStruct((N, D), jnp.float32),
               jax.ShapeDtypeStruct((N, D), jnp.float32)],
)


def _tc_last_body(s0_ref, s1_ref, u_ref, w_ref, gw_ref, gb_ref, x_ref):
    sb = s0_ref[...] + s1_ref[...]
    h = _leaky(_mm_t(sb, w_ref[...]))
    x_ref[...] = _leaky(_mm_t(h, gw_ref[...]) + gb_ref[...] + u_ref[...])


_tc_last = pl.pallas_call(
    _tc_last_body,
    grid=(GRID,),
    in_specs=[_row_spec, _row_spec, _row_spec, _w_spec, _w_spec, _b_spec],
    out_specs=_row_spec,
    out_shape=jax.ShapeDtypeStruct((N, D), jnp.float32),
)


@jax.jit
def _run(features, id_embedding, edge_index, preference,
         W1, W2, W3, W4, L1w, L1b, L2w, L2b, L3w, L3b, L4w, L4b,
         G1w, G1b, G2w, G2b, G3w, G3b, G4w, G4b):
    E = edge_index.shape[1]
    total_chunks = (E + CHUNK - 1) // CHUNK
    per_tile = (total_chunks + NS - 1) // NS
    cpt0 = int(round(per_tile * 0.62))
    cpt1 = per_tile - cpt0
    EP = (cpt0 + cpt1) * NS * CHUNK
    pad = EP - E
    src = jnp.concatenate(
        [edge_index[0].astype(jnp.int32), jnp.zeros((pad,), jnp.int32)])
    dst = jnp.concatenate(
        [edge_index[1].astype(jnp.int32), jnp.full((pad,), N, jnp.int32)])
    zeros = jnp.zeros((ACC_ROWS, D), jnp.float32)

    seg_sum = _make_seg_sum(cpt0, cpt1)

    x0 = jnp.concatenate([preference, features], axis=0)
    x, u = _tc_first(x0, id_embedding, L1w, L1b.reshape(1, D))

    layers = [(W1, G1w, G1b, L2w, L2b),
              (W2, G2w, G2b, L3w, L3b),
              (W3, G3w, G3b, L4w, L4b)]
    for (W, Gw, Gb, Lw, Lb) in layers:
        parts = seg_sum(x, src, dst, zeros)
        x, u = _tc_mid(parts[0, :N], parts[1, :N], u, id_embedding, W, Gw,
                       Gb.reshape(1, D), Lw, Lb.reshape(1, D))

    parts = seg_sum(x, src, dst, zeros)
    return _tc_last(parts[0, :N], parts[1, :N], u, W4, G4w, G4b.reshape(1, D))


def kernel(features, id_embedding, edge_index, preference,
           W1, W2, W3, W4, L1w, L1b, L2w, L2b, L3w, L3b, L4w, L4b,
           G1w, G1b, G2w, G2b, G3w, G3b, G4w, G4b):
    return _run(features, id_embedding, edge_index, preference,
                W1, W2, W3, W4, L1w, L1b, L2w, L2b, L3w, L3b, L4w, L4b,
                G1w, G1b, G2w, G2b, G3w, G3b, G4w, G4b)


# speedup vs baseline: 1.0737x; 1.0737x over previous
import functools

import jax
import jax.numpy as jnp
from jax import lax
from jax.experimental import pallas as pl
from jax.experimental.pallas import tpu as pltpu
from jax.experimental.pallas import tpu_sc as plsc

N = 10000
D = 128
NC = 2
NS = 16
CHUNK = 128
ACC_ROWS = 10112
BLK = 1000
GRID = N // BLK


def _leaky(v):
    return jnp.where(v >= 0, v, 0.01 * v)


def _mm_t(a, b):
    return lax.dot_general(a, b, (((1,), (1,)), ((), ())),
                           preferred_element_type=jnp.float32)


def _make_seg_sum(cpt0, cpt1):
    mesh = plsc.VectorSubcoreMesh(core_axis_name="c", subcore_axis_name="s")

    @functools.partial(
        pl.kernel,
        out_type=jax.ShapeDtypeStruct((NC, ACC_ROWS, D), jnp.float32),
        mesh=mesh,
        scratch_types=[
            pltpu.VMEM((CHUNK,), jnp.int32),
            pltpu.VMEM((CHUNK,), jnp.int32),
            pltpu.VMEM((CHUNK, D), jnp.float32),
            pltpu.VMEM_SHARED((ACC_ROWS, D), jnp.float32),
            pltpu.SemaphoreType.DMA,
        ],
    )
    def seg_sum(x_hbm, src_hbm, dst_hbm, zeros_hbm, out_hbm,
                sbuf, dbuf, rows, acc, gsem):
        c = lax.axis_index("c")
        s = lax.axis_index("s")
        cpt = jnp.where(c == 0, cpt0, cpt1)
        tbase = jnp.where(c == 0, s * cpt0, NS * cpt0 + s * cpt1)
        ebase = tbase * CHUNK

        zr = ACC_ROWS // NS
        pltpu.sync_copy(zeros_hbm.at[pl.ds(s * zr, zr)],
                        acc.at[pl.ds(s * zr, zr)])
        plsc.subcore_barrier()

        def body(i, carry):
            off = ebase + i * CHUNK
            pltpu.sync_copy(src_hbm.at[pl.ds(off, CHUNK)], sbuf)
            pltpu.sync_copy(dst_hbm.at[pl.ds(off, CHUNK)], dbuf)
            pltpu.async_copy(x_hbm.at[sbuf], rows, gsem).wait()
            pltpu.sync_copy(rows, acc.at[dbuf], add=True)
            return carry

        lax.fori_loop(0, cpt, body, 0)

        plsc.subcore_barrier()

        orow = ACC_ROWS // NS
        pltpu.sync_copy(acc.at[pl.ds(s * orow, orow)],
                        out_hbm.at[c, pl.ds(s * orow, orow)])

    return seg_sum


_row_spec = pl.BlockSpec((BLK, D), lambda i: (i, 0))
_w_spec = pl.BlockSpec((D, D), lambda i: (0, 0))
_b_spec = pl.BlockSpec((1, D), lambda i: (0, 0))


def _tc_first_body(x_ref, id_ref, lw_ref, lb_ref, x1_ref, u1_ref):
    xb = x_ref[...]
    ss = jnp.sum(xb * xb, axis=1, keepdims=True)
    xn = xb / jnp.maximum(jnp.sqrt(ss), 1e-12)
    x1_ref[...] = xn
    u1_ref[...] = _leaky(_mm_t(xn, lw_ref[...]) + lb_ref[...]) + id_ref[...]


_tc_first = pl.pallas_call(
    _tc_first_body,
    grid=(GRID,),
    in_specs=[_row_spec, _row_spec, _w_spec, _b_spec],
    out_specs=[_row_spec, _row_spec],
    out_shape=[jax.ShapeDtypeStruct((N, D), jnp.float32),
               jax.ShapeDtypeStruct((N, D), jnp.float32)],
)


def _tc_mid_body(s0_ref, s1_ref, u_ref, id_ref, w_ref, gw_ref, gb_ref,
                 lw_ref, lb_ref, x_ref, un_ref):
    sb = s0_ref[...] + s1_ref[...]
    h = _leaky(_mm_t(sb, w_ref[...]))
    x = _leaky(_mm_t(h, gw_ref[...]) + gb_ref[...] + u_ref[...])
    x_ref[...] = x
    un_ref[...] = _leaky(_mm_t(x, lw_ref[...]) + lb_ref[...]) + id_ref[...]


_tc_mid = pl.pallas_call(
    _tc_mid_body,
    grid=(GRID,),
    in_specs=[_row_spec, _row_spec, _row_spec, _row_spec,
              _w_spec, _w_spec, _b_spec, _w_spec, _b_spec],
    out_specs=[_row_spec, _row_spec],
    out_shape=[jax.ShapeDtypeStruct((N, D), jnp.float32),
               jax.ShapeDtypeStruct((N, D), jnp.float32)],
)


def _tc_last_body(s0_ref, s1_ref, u_ref, w_ref, gw_ref, gb_ref, x_ref):
    sb = s0_ref[...] + s1_ref[...]
    h = _leaky(_mm_t(sb, w_ref[...]))
    x_ref[...] = _leaky(_mm_t(h, gw_ref[...]) + gb_ref[...] + u_ref[...])


_tc_last = pl.pallas_call(
    _tc_last_body,
    grid=(GRID,),
    in_specs=[_row_spec, _row_spec, _row_spec, _w_spec, _w_spec, _b_spec],
    out_specs=_row_spec,
    out_shape=jax.ShapeDtypeStruct((N, D), jnp.float32),
)


@jax.jit
def _run(features, id_embedding, edge_index, preference,
         W1, W2, W3, W4, L1w, L1b, L2w, L2b, L3w, L3b, L4w, L4b,
         G1w, G1b, G2w, G2b, G3w, G3b, G4w, G4b):
    E = edge_index.shape[1]
    total_chunks = (E + CHUNK - 1) // CHUNK
    per_tile = (total_chunks + NS - 1) // NS
    cpt0 = int(round(per_tile * 0.57))
    cpt1 = per_tile - cpt0
    EP = (cpt0 + cpt1) * NS * CHUNK
    pad = EP - E
    src = jnp.concatenate(
        [edge_index[0].astype(jnp.int32), jnp.zeros((pad,), jnp.int32)])
    dst = jnp.concatenate(
        [edge_index[1].astype(jnp.int32), jnp.full((pad,), N, jnp.int32)])
    zeros = jnp.zeros((ACC_ROWS, D), jnp.float32)

    seg_sum = _make_seg_sum(cpt0, cpt1)

    x0 = jnp.concatenate([preference, features], axis=0)
    x, u = _tc_first(x0, id_embedding, L1w, L1b.reshape(1, D))

    layers = [(W1, G1w, G1b, L2w, L2b),
              (W2, G2w, G2b, L3w, L3b),
              (W3, G3w, G3b, L4w, L4b)]
    for (W, Gw, Gb, Lw, Lb) in layers:
        parts = seg_sum(x, src, dst, zeros)
        x, u = _tc_mid(parts[0, :N], parts[1, :N], u, id_embedding, W, Gw,
                       Gb.reshape(1, D), Lw, Lb.reshape(1, D))

    parts = seg_sum(x, src, dst, zeros)
    return _tc_last(parts[0, :N], parts[1, :N], u, W4, G4w, G4b.reshape(1, D))


def kernel(features, id_embedding, edge_index, preference,
           W1, W2, W3, W4, L1w, L1b, L2w, L2b, L3w, L3b, L4w, L4b,
           G1w, G1b, G2w, G2b, G3w, G3b, G4w, G4b):
    return _run(features, id_embedding, edge_index, preference,
                W1, W2, W3, W4, L1w, L1b, L2w, L2b, L3w, L3b, L4w, L4b,
                G1w, G1b, G2w, G2b, G3w, G3b, G4w, G4b)


# speedup vs baseline: 1.0862x; 1.0117x over previous
import functools

import jax
import jax.numpy as jnp
from jax import lax
from jax.experimental import pallas as pl
from jax.experimental.pallas import tpu as pltpu
from jax.experimental.pallas import tpu_sc as plsc

N = 10000
D = 128
NC = 2
NS = 16
CHUNK = 128
ACC_ROWS = 10112
BLK = 1000
GRID = N // BLK


def _leaky(v):
    return jnp.where(v >= 0, v, 0.01 * v)


def _mm_t(a, b):
    return lax.dot_general(a, b, (((1,), (1,)), ((), ())),
                           preferred_element_type=jnp.float32)


def _make_seg_sum(cpt0, cpt1):
    mesh = plsc.VectorSubcoreMesh(core_axis_name="c", subcore_axis_name="s")

    @functools.partial(
        pl.kernel,
        out_type=jax.ShapeDtypeStruct((NC, ACC_ROWS, D), jnp.float32),
        mesh=mesh,
        scratch_types=[
            pltpu.VMEM((CHUNK,), jnp.int32),
            pltpu.VMEM((CHUNK,), jnp.int32),
            pltpu.VMEM((CHUNK, D), jnp.float32),
            pltpu.VMEM_SHARED((ACC_ROWS, D), jnp.float32),
            pltpu.SemaphoreType.DMA,
        ],
    )
    def seg_sum(x_hbm, src_hbm, dst_hbm, zeros_hbm, out_hbm,
                sbuf, dbuf, rows, acc, gsem):
        c = lax.axis_index("c")
        s = lax.axis_index("s")
        cpt = jnp.where(c == 0, cpt0, cpt1)
        tbase = jnp.where(c == 0, s * cpt0, NS * cpt0 + s * cpt1)
        ebase = tbase * CHUNK

        zr = ACC_ROWS // NS
        pltpu.sync_copy(zeros_hbm.at[pl.ds(s * zr, zr)],
                        acc.at[pl.ds(s * zr, zr)])
        plsc.subcore_barrier()

        def body(i, carry):
            off = ebase + i * CHUNK
            pltpu.sync_copy(src_hbm.at[pl.ds(off, CHUNK)], sbuf)
            pltpu.sync_copy(dst_hbm.at[pl.ds(off, CHUNK)], dbuf)
            pltpu.async_copy(x_hbm.at[sbuf], rows, gsem).wait()
            pltpu.sync_copy(rows, acc.at[dbuf], add=True)
            return carry

        lax.fori_loop(0, cpt, body, 0)

        plsc.subcore_barrier()

        orow = ACC_ROWS // NS
        pltpu.sync_copy(acc.at[pl.ds(s * orow, orow)],
                        out_hbm.at[c, pl.ds(s * orow, orow)])

    return seg_sum


_row_spec = pl.BlockSpec((BLK, D), lambda i: (i, 0))
_w_spec = pl.BlockSpec((D, D), lambda i: (0, 0))
_b_spec = pl.BlockSpec((1, D), lambda i: (0, 0))


def _tc_first_body(x_ref, id_ref, lw_ref, lb_ref, x1_ref, u1_ref):
    xb = x_ref[...]
    ss = jnp.sum(xb * xb, axis=1, keepdims=True)
    xn = xb / jnp.maximum(jnp.sqrt(ss), 1e-12)
    x1_ref[...] = xn
    u1_ref[...] = _leaky(_mm_t(xn, lw_ref[...]) + lb_ref[...]) + id_ref[...]


_tc_first = pl.pallas_call(
    _tc_first_body,
    grid=(GRID,),
    in_specs=[_row_spec, _row_spec, _w_spec, _b_spec],
    out_specs=[_row_spec, _row_spec],
    out_shape=[jax.ShapeDtypeStruct((N, D), jnp.float32),
               jax.ShapeDtypeStruct((N, D), jnp.float32)],
)


def _tc_mid_body(s0_ref, s1_ref, u_ref, id_ref, w_ref, gw_ref, gb_ref,
                 lw_ref, lb_ref, x_ref, un_ref):
    sb = s0_ref[...] + s1_ref[...]
    h = _leaky(_mm_t(sb, w_ref[...]))
    x = _leaky(_mm_t(h, gw_ref[...]) + gb_ref[...] + u_ref[...])
    x_ref[...] = x
    un_ref[...] = _leaky(_mm_t(x, lw_ref[...]) + lb_ref[...]) + id_ref[...]


_tc_mid = pl.pallas_call(
    _tc_mid_body,
    grid=(GRID,),
    in_specs=[_row_spec, _row_spec, _row_spec, _row_spec,
              _w_spec, _w_spec, _b_spec, _w_spec, _b_spec],
    out_specs=[_row_spec, _row_spec],
    out_shape=[jax.ShapeDtypeStruct((N, D), jnp.float32),
               jax.ShapeDtypeStruct((N, D), jnp.float32)],
)


def _tc_last_body(s0_ref, s1_ref, u_ref, w_ref, gw_ref, gb_ref, x_ref):
    sb = s0_ref[...] + s1_ref[...]
    h = _leaky(_mm_t(sb, w_ref[...]))
    x_ref[...] = _leaky(_mm_t(h, gw_ref[...]) + gb_ref[...] + u_ref[...])


_tc_last = pl.pallas_call(
    _tc_last_body,
    grid=(GRID,),
    in_specs=[_row_spec, _row_spec, _row_spec, _w_spec, _w_spec, _b_spec],
    out_specs=_row_spec,
    out_shape=jax.ShapeDtypeStruct((N, D), jnp.float32),
)


@jax.jit
def _run(features, id_embedding, edge_index, preference,
         W1, W2, W3, W4, L1w, L1b, L2w, L2b, L3w, L3b, L4w, L4b,
         G1w, G1b, G2w, G2b, G3w, G3b, G4w, G4b):
    E = edge_index.shape[1]
    total_chunks = (E + CHUNK - 1) // CHUNK
    per_tile = (total_chunks + NS - 1) // NS
    cpt0 = int(round(per_tile * 0.55))
    cpt1 = per_tile - cpt0
    EP = (cpt0 + cpt1) * NS * CHUNK
    pad = EP - E
    src = jnp.concatenate(
        [edge_index[0].astype(jnp.int32), jnp.zeros((pad,), jnp.int32)])
    dst = jnp.concatenate(
        [edge_index[1].astype(jnp.int32), jnp.full((pad,), N, jnp.int32)])
    zeros = jnp.zeros((ACC_ROWS, D), jnp.float32)

    seg_sum = _make_seg_sum(cpt0, cpt1)

    x0 = jnp.concatenate([preference, features], axis=0)
    x, u = _tc_first(x0, id_embedding, L1w, L1b.reshape(1, D))

    layers = [(W1, G1w, G1b, L2w, L2b),
              (W2, G2w, G2b, L3w, L3b),
              (W3, G3w, G3b, L4w, L4b)]
    for (W, Gw, Gb, Lw, Lb) in layers:
        parts = seg_sum(x, src, dst, zeros)
        x, u = _tc_mid(parts[0, :N], parts[1, :N], u, id_embedding, W, Gw,
                       Gb.reshape(1, D), Lw, Lb.reshape(1, D))

    parts = seg_sum(x, src, dst, zeros)
    return _tc_last(parts[0, :N], parts[1, :N], u, W4, G4w, G4b.reshape(1, D))


def kernel(features, id_embedding, edge_index, preference,
           W1, W2, W3, W4, L1w, L1b, L2w, L2b, L3w, L3b, L4w, L4b,
           G1w, G1b, G2w, G2b, G3w, G3b, G4w, G4b):
    return _run(features, id_embedding, edge_index, preference,
                W1, W2, W3, W4, L1w, L1b, L2w, L2b, L3w, L3b, L4w, L4b,
                G1w, G1b, G2w, G2b, G3w, G3b, G4w, G4b)


# speedup vs baseline: 1.1904x; 1.0959x over previous
import functools

import jax
import jax.numpy as jnp
from jax import lax
from jax.experimental import pallas as pl
from jax.experimental.pallas import tpu as pltpu
from jax.experimental.pallas import tpu_sc as plsc

N = 10000
D = 128
NC = 2
NS = 16
CHUNK = 128
ACC_ROWS = 10112
BLK = 1000
GRID = N // BLK


def _leaky(v):
    return jnp.where(v >= 0, v, 0.01 * v)


def _mm_t(a, b):
    return lax.dot_general(a, b, (((1,), (1,)), ((), ())),
                           preferred_element_type=jnp.float32)


def _make_seg_sum(cpt0, cpt1):
    mesh = plsc.VectorSubcoreMesh(core_axis_name="c", subcore_axis_name="s")

    @functools.partial(
        pl.kernel,
        out_type=jax.ShapeDtypeStruct((NC, ACC_ROWS, D), jnp.float32),
        mesh=mesh,
        scratch_types=[
            pltpu.VMEM((CHUNK,), jnp.int32),
            pltpu.VMEM((CHUNK,), jnp.int32),
            pltpu.VMEM((CHUNK, D), jnp.float32),
            pltpu.VMEM_SHARED((ACC_ROWS, D), jnp.float32),
            pltpu.SemaphoreType.DMA,
            pltpu.SemaphoreType.DMA,
            pltpu.SemaphoreType.DMA,
        ],
    )
    def seg_sum(x_hbm, src_hbm, dst_hbm, zeros_hbm, out_hbm,
                sbuf, dbuf, rows, acc, gsem, isem1, isem2):
        c = lax.axis_index("c")
        s = lax.axis_index("s")
        cpt = jnp.where(c == 0, cpt0, cpt1)
        tbase = jnp.where(c == 0, s * cpt0, NS * cpt0 + s * cpt1)
        ebase = tbase * CHUNK

        zr = ACC_ROWS // NS
        pltpu.sync_copy(zeros_hbm.at[pl.ds(s * zr, zr)],
                        acc.at[pl.ds(s * zr, zr)])
        plsc.subcore_barrier()

        def body(i, carry):
            off = ebase + i * CHUNK
            d1 = pltpu.async_copy(src_hbm.at[pl.ds(off, CHUNK)], sbuf, isem1)
            d2 = pltpu.async_copy(dst_hbm.at[pl.ds(off, CHUNK)], dbuf, isem2)
            d1.wait()
            d2.wait()
            pltpu.async_copy(x_hbm.at[sbuf], rows, gsem).wait()
            pltpu.sync_copy(rows, acc.at[dbuf], add=True)
            return carry

        lax.fori_loop(0, cpt, body, 0)

        plsc.subcore_barrier()

        orow = ACC_ROWS // NS
        pltpu.sync_copy(acc.at[pl.ds(s * orow, orow)],
                        out_hbm.at[c, pl.ds(s * orow, orow)])

    return seg_sum


_row_spec = pl.BlockSpec((BLK, D), lambda i: (i, 0))
_w_spec = pl.BlockSpec((D, D), lambda i: (0, 0))
_b_spec = pl.BlockSpec((1, D), lambda i: (0, 0))


def _tc_first_body(x_ref, id_ref, lw_ref, lb_ref, x1_ref, u1_ref):
    xb = x_ref[...]
    ss = jnp.sum(xb * xb, axis=1, keepdims=True)
    xn = xb / jnp.maximum(jnp.sqrt(ss), 1e-12)
    x1_ref[...] = xn
    u1_ref[...] = _leaky(_mm_t(xn, lw_ref[...]) + lb_ref[...]) + id_ref[...]


_tc_first = pl.pallas_call(
    _tc_first_body,
    grid=(GRID,),
    in_specs=[_row_spec, _row_spec, _w_spec, _b_spec],
    out_specs=[_row_spec, _row_spec],
    out_shape=[jax.ShapeDtypeStruct((N, D), jnp.float32),
               jax.ShapeDtypeStruct((N, D), jnp.float32)],
)


def _tc_mid_body(s0_ref, s1_ref, u_ref, id_ref, w_ref, gw_ref, gb_ref,
                 lw_ref, lb_ref, x_ref, un_ref):
    sb = s0_ref[...] + s1_ref[...]
    h = _leaky(_mm_t(sb, w_ref[...]))
    x = _leaky(_mm_t(h, gw_ref[...]) + gb_ref[...] + u_ref[...])
    x_ref[...] = x
    un_ref[...] = _leaky(_mm_t(x, lw_ref[...]) + lb_ref[...]) + id_ref[...]


_tc_mid = pl.pallas_call(
    _tc_mid_body,
    grid=(GRID,),
    in_specs=[_row_spec, _row_spec, _row_spec, _row_spec,
              _w_spec, _w_spec, _b_spec, _w_spec, _b_spec],
    out_specs=[_row_spec, _row_spec],
    out_shape=[jax.ShapeDtypeStruct((N, D), jnp.float32),
               jax.ShapeDtypeStruct((N, D), jnp.float32)],
)


def _tc_last_body(s0_ref, s1_ref, u_ref, w_ref, gw_ref, gb_ref, x_ref):
    sb = s0_ref[...] + s1_ref[...]
    h = _leaky(_mm_t(sb, w_ref[...]))
    x_ref[...] = _leaky(_mm_t(h, gw_ref[...]) + gb_ref[...] + u_ref[...])


_tc_last = pl.pallas_call(
    _tc_last_body,
    grid=(GRID,),
    in_specs=[_row_spec, _row_spec, _row_spec, _w_spec, _w_spec, _b_spec],
    out_specs=_row_spec,
    out_shape=jax.ShapeDtypeStruct((N, D), jnp.float32),
)


@jax.jit
def _run(features, id_embedding, edge_index, preference,
         W1, W2, W3, W4, L1w, L1b, L2w, L2b, L3w, L3b, L4w, L4b,
         G1w, G1b, G2w, G2b, G3w, G3b, G4w, G4b):
    E = edge_index.shape[1]
    total_chunks = (E + CHUNK - 1) // CHUNK
    per_tile = (total_chunks + NS - 1) // NS
    cpt0 = int(round(per_tile * 0.55))
    cpt1 = per_tile - cpt0
    EP = (cpt0 + cpt1) * NS * CHUNK
    pad = EP - E
    src = jnp.concatenate(
        [edge_index[0].astype(jnp.int32), jnp.zeros((pad,), jnp.int32)])
    dst = jnp.concatenate(
        [edge_index[1].astype(jnp.int32), jnp.full((pad,), N, jnp.int32)])
    zeros = jnp.zeros((ACC_ROWS, D), jnp.float32)

    seg_sum = _make_seg_sum(cpt0, cpt1)

    x0 = jnp.concatenate([preference, features], axis=0)
    x, u = _tc_first(x0, id_embedding, L1w, L1b.reshape(1, D))

    layers = [(W1, G1w, G1b, L2w, L2b),
              (W2, G2w, G2b, L3w, L3b),
              (W3, G3w, G3b, L4w, L4b)]
    for (W, Gw, Gb, Lw, Lb) in layers:
        parts = seg_sum(x, src, dst, zeros)
        x, u = _tc_mid(parts[0, :N], parts[1, :N], u, id_embedding, W, Gw,
                       Gb.reshape(1, D), Lw, Lb.reshape(1, D))

    parts = seg_sum(x, src, dst, zeros)
    return _tc_last(parts[0, :N], parts[1, :N], u, W4, G4w, G4b.reshape(1, D))


def kernel(features, id_embedding, edge_index, preference,
           W1, W2, W3, W4, L1w, L1b, L2w, L2b, L3w, L3b, L4w, L4b,
           G1w, G1b, G2w, G2b, G3w, G3b, G4w, G4b):
    return _run(features, id_embedding, edge_index, preference,
                W1, W2, W3, W4, L1w, L1b, L2w, L2b, L3w, L3b, L4w, L4b,
                G1w, G1b, G2w, G2b, G3w, G3b, G4w, G4b)
